# baseline (speedup 1.0000x reference)
import jax
import jax.numpy as jnp
import numpy as np
from jax.experimental import pallas as pl

_HI = jax.lax.Precision.HIGHEST


def _dot_mxu(a, b):
    return jnp.dot(a.astype(jnp.bfloat16), b.astype(jnp.bfloat16),
                   preferred_element_type=jnp.float32)

def _split3(v):
    f32, bf16 = jnp.float32, jnp.bfloat16
    hi = v.astype(bf16)
    r1 = v - hi.astype(f32)
    mid = r1.astype(bf16)
    lo = (r1 - mid.astype(f32)).astype(bf16)
    return hi, mid, lo


def _sel_dot(mask_bf16, v):
    f32 = jnp.float32
    hi, mid, lo = _split3(v)
    return (jnp.dot(mask_bf16, hi, preferred_element_type=f32)
            + jnp.dot(mask_bf16, mid, preferred_element_type=f32)
            + jnp.dot(mask_bf16, lo, preferred_element_type=f32))


_N1 = 64
_N = _N1 * _N1
_TAU = 0.05

_IA = np.arange(_N)[:, None]
_KA = np.arange(_N1)[None, :]
_R_NP = ((_IA // _N1) == _KA).astype(np.float32)
_T_NP = ((_IA % _N1) == _KA).astype(np.float32)


def _sinkhorn(ls):
    for it in range(20):
        axis = 1 if it % 2 == 0 else 0
        mx = jnp.max(ls, axis=axis, keepdims=True)
        ls = ls - (mx + jnp.log(jnp.sum(jnp.exp(ls - mx), axis=axis, keepdims=True)))
    return jnp.exp(ls)


def _gcn_net_kernel(ep_ref, cp_ref,
                    nW1_ref, nb1_ref, nW2_ref, nb2_ref,
                    sW0_ref, sb0_ref, gW0_ref, gb0_ref, kW0_ref, kb0_ref,
                    sW1_ref, sb1_ref, gW1_ref, gb1_ref, kW1_ref, kb1_ref,
                    cW_ref, cb_ref,
                    Rb_ref, T_ref, Rtb_ref, Ttb_ref, o_ref):
    f32 = jnp.float32
    ep = ep_ref[...]
    cp = cp_ref[...]

    Rb = Rb_ref[...]
    T = T_ref[...]
    Rtb = Rtb_ref[...]
    Ttb = Ttb_ref[...]

    def flat(v2d):
        z = _sel_dot(Rb, v2d)
        return jnp.sum(z * T, axis=1, keepdims=True)

    def to2d(vflat):
        return _sel_dot(Rtb, vflat * T)

    def node_feat(p):
        h = jnp.maximum(_dot_mxu(p[:, 1:6], nW1_ref[...]) + nb1_ref[...], 0.0)
        f = _dot_mxu(h, nW2_ref[...]) + nb2_ref[...]
        nrm = jnp.maximum(jnp.sqrt(jnp.sum(f * f, axis=1, keepdims=True)), 1e-8)
        return f / nrm

    f1n = node_feat(ep)
    f2n = node_feat(cp)
    dist2d = _dot_mxu(f1n, f2n.T)
    m2d = (ep[:, 6:7] == cp[:, 6:7].T).astype(f32)
    conf2d = (jnp.sqrt(ep[:, 7:8] * cp[:, 7:8].T)
              + jnp.sqrt(ep[:, 8:9] * cp[:, 8:9].T))
    naff2d = m2d * conf2d * dist2d

    bf16 = jnp.bfloat16
    naff = flat(naff2d)
    mflat = jnp.sum(jnp.dot(Rb, m2d.astype(bf16), preferred_element_type=f32) * T,
                    axis=1, keepdims=True)

    Sm = jnp.sum(m2d)
    Rm2 = jnp.sum(m2d, axis=1, keepdims=True)
    Cm2 = jnp.sum(m2d, axis=0, keepdims=True)
    nz2 = (naff2d != 0.0).astype(f32)
    deg2d = m2d * (Sm - Rm2 - Cm2 + 1.0) + nz2 + 1.0
    dinv = flat(1.0 / jnp.sqrt(deg2d))
    nzp1 = jnp.sum(jnp.dot(Rb, (nz2 + 1.0).astype(bf16), preferred_element_type=f32) * T,
                   axis=1, keepdims=True)

    def collapsed_gcn(xw):
        y = dinv * xw
        u = mflat * y
        F = u.shape[1]
        Sy = jnp.sum(u, axis=0, keepdims=True)
        u3 = u.reshape(_N1, _N1, F)
        Ry_col = jnp.broadcast_to(jnp.sum(u3, axis=1, keepdims=True),
                                  (_N1, _N1, F)).reshape(_N, F)
        Cy_col = jnp.broadcast_to(jnp.sum(u3, axis=0, keepdims=True),
                                  (_N1, _N1, F)).reshape(_N, F)
        msg = mflat * (Sy - Ry_col - Cy_col + u) + nzp1 * y
        return dinv * msg

    def ngm_layer(x, sW_ref, sb_ref, gW_ref, gb_ref, kW_ref, kb_ref):
        if gW_ref.shape[0] == 1:
            xw = x * gW_ref[...]
            x_self = x * sW_ref[...] + sb_ref[...]
        else:
            xw = _dot_mxu(x, gW_ref[...])
            x_self = _dot_mxu(x, sW_ref[...]) + sb_ref[...]
        x_neigh = collapsed_gcn(xw) + gb_ref[...]
        x_out = x_self + x_neigh
        skf = _dot_mxu(x_out, kW_ref[...]) + kb_ref[...]
        sk = _sinkhorn(to2d(skf) / f32(_TAU))
        return x_out + flat(sk)

    x = ngm_layer(naff, sW0_ref, sb0_ref, gW0_ref, gb0_ref, kW0_ref, kb0_ref)
    x = ngm_layer(x, sW1_ref, sb1_ref, gW1_ref, gb1_ref, kW1_ref, kb1_ref)

    scores = _dot_mxu(x, cW_ref[...]) + cb_ref[...]
    s = _sel_dot(Ttb, scores * Rb.astype(f32))
    o_ref[...] = _sinkhorn(s / f32(_TAU))


def kernel(ego_preds, cav_preds, params):
    p = params
    args = [
        ego_preds[0], cav_preds[0],
        p['node_W1'], p['node_b1'].reshape(1, -1),
        p['node_W2'], p['node_b2'].reshape(1, -1),
        p['l0_self_W'], p['l0_self_b'].reshape(1, -1),
        p['l0_gcn_W'], p['l0_gcn_b'].reshape(1, -1),
        p['l0_sk_W'], p['l0_sk_b'].reshape(1, -1),
        p['l1_self_W'], p['l1_self_b'].reshape(1, -1),
        p['l1_gcn_W'], p['l1_gcn_b'].reshape(1, -1),
        p['l1_sk_W'], p['l1_sk_b'].reshape(1, -1),
        p['cls_W'], p['cls_b'].reshape(1, -1),
    ]
    args += [
        jnp.asarray(_R_NP, jnp.bfloat16),
        jnp.asarray(_T_NP, jnp.float32),
        jnp.asarray(_R_NP.T, jnp.bfloat16),
        jnp.asarray(_T_NP.T, jnp.bfloat16),
    ]
    return pl.pallas_call(
        _gcn_net_kernel,
        out_shape=jax.ShapeDtypeStruct((_N1, _N1), jnp.float32),
    )(*args)


# speedup vs baseline: 1.1362x; 1.1362x over previous
import jax
import jax.numpy as jnp
from jax.experimental import pallas as pl

_HI = jax.lax.Precision.HIGHEST


def _dot_mxu(a, b):
    return jnp.dot(a.astype(jnp.bfloat16), b.astype(jnp.bfloat16),
                   preferred_element_type=jnp.float32)

def _split3(v):
    f32, bf16 = jnp.float32, jnp.bfloat16
    hi = v.astype(bf16)
    r1 = v - hi.astype(f32)
    mid = r1.astype(bf16)
    lo = (r1 - mid.astype(f32)).astype(bf16)
    return hi, mid, lo


def _sel_dot(mask_bf16, v):
    f32 = jnp.float32
    hi, mid, lo = _split3(v)
    return (jnp.dot(mask_bf16, hi, preferred_element_type=f32)
            + jnp.dot(mask_bf16, mid, preferred_element_type=f32)
            + jnp.dot(mask_bf16, lo, preferred_element_type=f32))


_N1 = 64
_N = _N1 * _N1
_TAU = 0.05


def _sinkhorn(ls):
    for it in range(20):
        axis = 1 if it % 2 == 0 else 0
        mx = jnp.max(ls, axis=axis, keepdims=True)
        ls = ls - (mx + jnp.log(jnp.sum(jnp.exp(ls - mx), axis=axis, keepdims=True)))
    return jnp.exp(ls)


def _gcn_net_kernel(ep_ref, cp_ref,
                    nW1_ref, nb1_ref, nW2_ref, nb2_ref,
                    sW0_ref, sb0_ref, gW0_ref, gb0_ref, kW0_ref, kb0_ref,
                    sW1_ref, sb1_ref, gW1_ref, gb1_ref, kW1_ref, kb1_ref,
                    cW_ref, cb_ref, o_ref):
    f32 = jnp.float32
    ep = ep_ref[...]
    cp = cp_ref[...]

    bf16 = jnp.bfloat16
    ii = jax.lax.broadcasted_iota(jnp.int32, (_N, _N1), 0)
    kk = jax.lax.broadcasted_iota(jnp.int32, (_N, _N1), 1)
    Rb = (ii // _N1 == kk).astype(bf16)
    T = (ii % _N1 == kk).astype(f32)
    aa = jax.lax.broadcasted_iota(jnp.int32, (_N1, _N), 0)
    jj = jax.lax.broadcasted_iota(jnp.int32, (_N1, _N), 1)
    Rtb = (jj // _N1 == aa).astype(bf16)
    Ttb = (jj % _N1 == aa).astype(bf16)

    def flat(v2d):
        z = _sel_dot(Rb, v2d)
        return jnp.sum(z * T, axis=1, keepdims=True)

    def to2d(vflat):
        return _sel_dot(Rtb, vflat * T)

    def node_feat(p):
        h = jnp.maximum(_dot_mxu(p[:, 1:6], nW1_ref[...]) + nb1_ref[...], 0.0)
        f = _dot_mxu(h, nW2_ref[...]) + nb2_ref[...]
        nrm = jnp.maximum(jnp.sqrt(jnp.sum(f * f, axis=1, keepdims=True)), 1e-8)
        return f / nrm

    f1n = node_feat(ep)
    f2n = node_feat(cp)
    dist2d = _dot_mxu(f1n, f2n.T)
    m2d = (ep[:, 6:7] == cp[:, 6:7].T).astype(f32)
    conf2d = (jnp.sqrt(ep[:, 7:8] * cp[:, 7:8].T)
              + jnp.sqrt(ep[:, 8:9] * cp[:, 8:9].T))
    naff2d = m2d * conf2d * dist2d

    bf16 = jnp.bfloat16
    naff = flat(naff2d)
    mflat = jnp.sum(jnp.dot(Rb, m2d.astype(bf16), preferred_element_type=f32) * T,
                    axis=1, keepdims=True)

    Sm = jnp.sum(m2d)
    Rm2 = jnp.sum(m2d, axis=1, keepdims=True)
    Cm2 = jnp.sum(m2d, axis=0, keepdims=True)
    nz2 = (naff2d != 0.0).astype(f32)
    deg2d = m2d * (Sm - Rm2 - Cm2 + 1.0) + nz2 + 1.0
    dinv = flat(1.0 / jnp.sqrt(deg2d))
    nzp1 = jnp.sum(jnp.dot(Rb, (nz2 + 1.0).astype(bf16), preferred_element_type=f32) * T,
                   axis=1, keepdims=True)

    def collapsed_gcn(xw):
        y = dinv * xw
        u = mflat * y
        F = u.shape[1]
        Sy = jnp.sum(u, axis=0, keepdims=True)
        u3 = u.reshape(_N1, _N1, F)
        Ry_col = jnp.broadcast_to(jnp.sum(u3, axis=1, keepdims=True),
                                  (_N1, _N1, F)).reshape(_N, F)
        Cy_col = jnp.broadcast_to(jnp.sum(u3, axis=0, keepdims=True),
                                  (_N1, _N1, F)).reshape(_N, F)
        msg = mflat * (Sy - Ry_col - Cy_col + u) + nzp1 * y
        return dinv * msg

    def ngm_layer(x, sW_ref, sb_ref, gW_ref, gb_ref, kW_ref, kb_ref):
        if gW_ref.shape[0] == 1:
            xw = x * gW_ref[...]
            x_self = x * sW_ref[...] + sb_ref[...]
        else:
            xw = _dot_mxu(x, gW_ref[...])
            x_self = _dot_mxu(x, sW_ref[...]) + sb_ref[...]
        x_neigh = collapsed_gcn(xw) + gb_ref[...]
        x_out = x_self + x_neigh
        skf = _dot_mxu(x_out, kW_ref[...]) + kb_ref[...]
        sk = _sinkhorn(to2d(skf) / f32(_TAU))
        return x_out + flat(sk)

    x = ngm_layer(naff, sW0_ref, sb0_ref, gW0_ref, gb0_ref, kW0_ref, kb0_ref)
    x = ngm_layer(x, sW1_ref, sb1_ref, gW1_ref, gb1_ref, kW1_ref, kb1_ref)

    scores = _dot_mxu(x, cW_ref[...]) + cb_ref[...]
    s = _sel_dot(Ttb, scores * Rb.astype(f32))
    o_ref[...] = _sinkhorn(s / f32(_TAU))


def kernel(ego_preds, cav_preds, params):
    p = params
    args = [
        ego_preds[0], cav_preds[0],
        p['node_W1'], p['node_b1'].reshape(1, -1),
        p['node_W2'], p['node_b2'].reshape(1, -1),
        p['l0_self_W'], p['l0_self_b'].reshape(1, -1),
        p['l0_gcn_W'], p['l0_gcn_b'].reshape(1, -1),
        p['l0_sk_W'], p['l0_sk_b'].reshape(1, -1),
        p['l1_self_W'], p['l1_self_b'].reshape(1, -1),
        p['l1_gcn_W'], p['l1_gcn_b'].reshape(1, -1),
        p['l1_sk_W'], p['l1_sk_b'].reshape(1, -1),
        p['cls_W'], p['cls_b'].reshape(1, -1),
    ]
    return pl.pallas_call(
        _gcn_net_kernel,
        out_shape=jax.ShapeDtypeStruct((_N1, _N1), jnp.float32),
    )(*args)
